# Initial kernel scaffold; baseline (speedup 1.0000x reference)
import jax
import jax.numpy as jnp
from jax.experimental import pallas as pl


def kernel(x, edge_index, edge_attr, W_in, b_in, We1_0, be1_0, We2_0, be2_0, root_0, bconv_0, We1_1, be1_1, We2_1, be2_1, root_1, bconv_1, We1_2, be1_2, We2_2, be2_2, root_2, bconv_2, W_out, b_out, Wd1, bd1, Wd2, bd2):
    raise NotImplementedError("write your pallas kernel here")


# speedup vs baseline: 3.2035x; 3.2035x over previous
import jax
import jax.numpy as jnp
from jax import lax
from jax.experimental import pallas as pl
from jax.experimental.pallas import tpu as pltpu
from jax.experimental.pallas import tpu_sc as plsc

N = 10000
E = 160000
FN = 128
FE = 16
H = 16
Z = 32

NC = 2
NS = 16
NW = NC * NS
CH = 128
NPAD = 10240
EPAD = 163840
NCH = EPAD // (NW * CH)
ZR = NPAD // NS

_f32 = jnp.float32


def _gather_body(h_hbm, src_hbm, g_hbm, idx_v, rows_v, sem):
    wid = lax.axis_index("c") * NS + lax.axis_index("s")
    base = wid * NCH
    pltpu.sync_copy(src_hbm.at[pl.ds(base, NCH)], idx_v)
    descs = [pltpu.async_copy(h_hbm.at[idx_v.at[j]], rows_v.at[j], sem)
             for j in range(NCH)]
    for d in descs:
        d.wait()
    pltpu.sync_copy(rows_v, g_hbm.at[pl.ds(base, NCH)])


def _scatter_body(msg_hbm, dst_hbm, zeros_hbm, out_hbm, idx_v, rows_v, zbuf,
                  sem, acc):
    c = lax.axis_index("c")
    s = lax.axis_index("s")
    base = (c * NS + s) * NCH
    pltpu.sync_copy(zeros_hbm.at[pl.ds(s * ZR, ZR)], zbuf)
    pltpu.sync_copy(zbuf, acc.at[pl.ds(s * ZR, ZR)])
    plsc.subcore_barrier()
    pltpu.sync_copy(dst_hbm.at[pl.ds(base, NCH)], idx_v)
    pltpu.sync_copy(msg_hbm.at[pl.ds(base, NCH)], rows_v)
    descs = [pltpu.async_copy(rows_v.at[j], acc.at[idx_v.at[j]], sem, add=True)
             for j in range(NCH)]
    for d in descs:
        d.wait()
    plsc.subcore_barrier()
    pltpu.sync_copy(acc.at[pl.ds(s * ZR, ZR)], zbuf)
    pltpu.sync_copy(zbuf, out_hbm.at[c, pl.ds(s * ZR, ZR)])


def _count_body(dst_hbm, zeros_hbm, ones_hbm, out_hbm, idx_v, obuf, zbuf,
                sem, acc):
    c = lax.axis_index("c")
    s = lax.axis_index("s")
    base = (c * NS + s) * NCH
    pltpu.sync_copy(zeros_hbm.at[pl.ds(s * ZR, ZR)], zbuf)
    pltpu.sync_copy(zbuf, acc.at[pl.ds(s * ZR, ZR)])
    pltpu.sync_copy(ones_hbm, obuf)
    plsc.subcore_barrier()
    pltpu.sync_copy(dst_hbm.at[pl.ds(base, NCH)], idx_v)
    descs = [pltpu.async_copy(obuf, acc.at[idx_v.at[j]], sem, add=True)
             for j in range(NCH)]
    for d in descs:
        d.wait()
    plsc.subcore_barrier()
    pltpu.sync_copy(acc.at[pl.ds(s * ZR, ZR)], zbuf)
    pltpu.sync_copy(zbuf, out_hbm.at[c, pl.ds(s * ZR, ZR)])


import functools


@functools.cache
def _sc_kernels():
    mesh = plsc.VectorSubcoreMesh(
        core_axis_name="c", subcore_axis_name="s",
        num_cores=NC, num_subcores=NS)
    params = pltpu.CompilerParams(use_tc_tiling_on_sc=False)
    gather = pl.kernel(
        _gather_body,
        out_type=jax.ShapeDtypeStruct((EPAD // CH, CH, H), _f32),
        mesh=mesh,
        scratch_types=[
            pltpu.VMEM((NCH, CH), jnp.int32),
            pltpu.VMEM((NCH, CH, H), _f32),
            pltpu.SemaphoreType.DMA,
        ],
        compiler_params=params,
    )
    scatter = pl.kernel(
        _scatter_body,
        out_type=jax.ShapeDtypeStruct((NC, NPAD, H), _f32),
        mesh=mesh,
        scratch_types=[
            pltpu.VMEM((NCH, CH), jnp.int32),
            pltpu.VMEM((NCH, CH, H), _f32),
            pltpu.VMEM((ZR, H), _f32),
            pltpu.SemaphoreType.DMA,
            pltpu.VMEM_SHARED((NPAD, H), _f32),
        ],
        compiler_params=params,
    )
    count = pl.kernel(
        _count_body,
        out_type=jax.ShapeDtypeStruct((NC, NPAD, H), _f32),
        mesh=mesh,
        scratch_types=[
            pltpu.VMEM((NCH, CH), jnp.int32),
            pltpu.VMEM((CH, H), _f32),
            pltpu.VMEM((ZR, H), _f32),
            pltpu.SemaphoreType.DMA,
            pltpu.VMEM_SHARED((NPAD, H), _f32),
        ],
        compiler_params=params,
    )
    return gather, scatter, count


TN = 2048

def _h0_body(x_ref, w_ref, b_ref, o_ref):
    o_ref[...] = jnp.maximum(
        jnp.dot(x_ref[...], w_ref[...], preferred_element_type=_f32)
        + b_ref[...], 0.0)


_h0 = pl.pallas_call(
    _h0_body,
    grid=(NPAD // TN,),
    in_specs=[
        pl.BlockSpec((TN, FN), lambda i: (i, 0)),
        pl.BlockSpec((FN, H), lambda i: (0, 0)),
        pl.BlockSpec((1, H), lambda i: (0, 0)),
    ],
    out_specs=pl.BlockSpec((TN, H), lambda i: (i, 0)),
    out_shape=jax.ShapeDtypeStruct((NPAD, H), _f32),
)


TE = 2048

def _edge_body(ea_ref, g_ref, we1_ref, be1_ref, we2p_ref, be2r_ref, r_ref,
               s_ref, o_ref):
    u = jnp.maximum(
        jnp.dot(ea_ref[...], we1_ref[...], preferred_element_type=_f32)
        + be1_ref[...], 0.0)
    g = g_ref[...]
    a = jnp.dot(g, we2p_ref[...], preferred_element_type=_f32)
    ue = jnp.dot(u, r_ref[...], preferred_element_type=_f32)
    o_ref[...] = (
        jnp.dot(ue * a, s_ref[...], preferred_element_type=_f32)
        + jnp.dot(g, be2r_ref[...], preferred_element_type=_f32))


_edge = pl.pallas_call(
    _edge_body,
    grid=(EPAD // TE,),
    in_specs=[
        pl.BlockSpec((TE, FE), lambda i: (i, 0)),
        pl.BlockSpec((TE, H), lambda i: (i, 0)),
        pl.BlockSpec((FE, H), lambda i: (0, 0)),
        pl.BlockSpec((1, H), lambda i: (0, 0)),
        pl.BlockSpec((H, H * H), lambda i: (0, 0)),
        pl.BlockSpec((H, H), lambda i: (0, 0)),
        pl.BlockSpec((H, H * H), lambda i: (0, 0)),
        pl.BlockSpec((H * H, H), lambda i: (0, 0)),
    ],
    out_specs=pl.BlockSpec((TE, H), lambda i: (i, 0)),
    out_shape=jax.ShapeDtypeStruct((EPAD, H), _f32),
)


def _node_body(p0_ref, p1_ref, c0_ref, c1_ref, h_ref, root_ref, b_ref, o_ref):
    cw = jnp.maximum(c0_ref[...][0] + c1_ref[...][0], 1.0)
    agg = (p0_ref[...][0] + p1_ref[...][0]) / cw
    o_ref[...] = jnp.maximum(
        agg + jnp.dot(h_ref[...], root_ref[...], preferred_element_type=_f32)
        + b_ref[...], 0.0)


_node = pl.pallas_call(
    _node_body,
    grid=(NPAD // TN,),
    in_specs=[
        pl.BlockSpec((1, TN, H), lambda i: (0, i, 0)),
        pl.BlockSpec((1, TN, H), lambda i: (1, i, 0)),
        pl.BlockSpec((1, TN, H), lambda i: (0, i, 0)),
        pl.BlockSpec((1, TN, H), lambda i: (1, i, 0)),
        pl.BlockSpec((TN, H), lambda i: (i, 0)),
        pl.BlockSpec((H, H), lambda i: (0, 0)),
        pl.BlockSpec((1, H), lambda i: (0, 0)),
    ],
    out_specs=pl.BlockSpec((TN, H), lambda i: (i, 0)),
    out_shape=jax.ShapeDtypeStruct((NPAD, H), _f32),
)


TD = 2000

def _dec_body(h_ref, wo_ref, bo_ref, w1_ref, b1_ref, w2_ref, b2_ref, o_ref):
    z = (jnp.dot(h_ref[...], wo_ref[...], preferred_element_type=_f32)
         + bo_ref[...])
    d = jnp.maximum(
        jnp.dot(z, w1_ref[...], preferred_element_type=_f32) + b1_ref[...],
        0.0)
    o_ref[...] = (jnp.dot(d, w2_ref[...], preferred_element_type=_f32)
                  + b2_ref[...])


_dec = pl.pallas_call(
    _dec_body,
    grid=(N // TD,),
    in_specs=[
        pl.BlockSpec((TD, H), lambda i: (i, 0)),
        pl.BlockSpec((H, Z), lambda i: (0, 0)),
        pl.BlockSpec((1, Z), lambda i: (0, 0)),
        pl.BlockSpec((Z, H), lambda i: (0, 0)),
        pl.BlockSpec((1, H), lambda i: (0, 0)),
        pl.BlockSpec((H, FN), lambda i: (0, 0)),
        pl.BlockSpec((1, FN), lambda i: (0, 0)),
    ],
    out_specs=pl.BlockSpec((TD, FN), lambda i: (i, 0)),
    out_shape=jax.ShapeDtypeStruct((N, FN), _f32),
)


def kernel(x, edge_index, edge_attr, W_in, b_in,
           We1_0, be1_0, We2_0, be2_0, root_0, bconv_0,
           We1_1, be1_1, We2_1, be2_1, root_1, bconv_1,
           We1_2, be1_2, We2_2, be2_2, root_2, bconv_2,
           W_out, b_out, Wd1, bd1, Wd2, bd2):
    src = edge_index[0]
    dst = edge_index[1]
    src_p = jnp.pad(src, (0, EPAD - E)).reshape(EPAD // CH, CH)
    dst_p = jnp.pad(dst, (0, EPAD - E), constant_values=N).reshape(
        EPAD // CH, CH)
    ea_p = jnp.pad(edge_attr, ((0, EPAD - E), (0, 0)))
    x_p = jnp.pad(x, ((0, NPAD - N), (0, 0)))
    zeros_np = jnp.zeros((NPAD, H), _f32)
    ones_ch = jnp.ones((CH, H), _f32)
    eye = jnp.eye(H, dtype=_f32)
    r_mat = jnp.repeat(eye, H, axis=1)
    s_mat = jnp.tile(eye, (H, 1))

    _gather, _scatter, _count = _sc_kernels()
    h = _h0(x_p, W_in, b_in.reshape(1, H))
    cnt = _count(dst_p, zeros_np, ones_ch)

    for We1, be1, We2, be2, root, bconv in (
            (We1_0, be1_0, We2_0, be2_0, root_0, bconv_0),
            (We1_1, be1_1, We2_1, be2_1, root_1, bconv_1),
            (We1_2, be1_2, We2_2, be2_2, root_2, bconv_2)):
        we2p = We2.reshape(H, H, H).transpose(1, 0, 2).reshape(H, H * H)
        be2r = be2.reshape(H, H)
        g3 = _gather(h, src_p)
        msg = _edge(ea_p, g3.reshape(EPAD, H), We1, be1.reshape(1, H),
                    we2p, be2r, r_mat, s_mat)
        parts = _scatter(msg.reshape(EPAD // CH, CH, H), dst_p, zeros_np)
        h = _node(parts, parts, cnt, cnt, h, root, bconv.reshape(1, H))

    return _dec(h, W_out, b_out.reshape(1, Z), Wd1, bd1.reshape(1, H),
                Wd2, bd2.reshape(1, FN))


# speedup vs baseline: 3.3401x; 1.0427x over previous
import jax
import jax.numpy as jnp
from jax import lax
from jax.experimental import pallas as pl
from jax.experimental.pallas import tpu as pltpu
from jax.experimental.pallas import tpu_sc as plsc

N = 10000
E = 160000
FN = 128
FE = 16
H = 16
Z = 32

NC = 2
NS = 16
NW = NC * NS
CH = 128
NPAD = 10240
EPAD = 163840
NCH = EPAD // (NW * CH)
ZR = NPAD // NS

_f32 = jnp.float32


def _gather_body(h_hbm, src_hbm, g_hbm, idx_v, rows_v, sem):
    wid = lax.axis_index("c") * NS + lax.axis_index("s")
    base = wid * NCH
    pltpu.sync_copy(src_hbm.at[pl.ds(base, NCH)], idx_v)
    descs = [pltpu.async_copy(h_hbm.at[idx_v.at[j]], rows_v.at[j], sem)
             for j in range(NCH)]
    for d in descs:
        d.wait()
    pltpu.sync_copy(rows_v, g_hbm.at[pl.ds(base, NCH)])


def _scatter_body(msg_hbm, dst_hbm, zeros_hbm, out_hbm, idx_v, rows_v, zbuf,
                  sem, acc):
    c = lax.axis_index("c")
    s = lax.axis_index("s")
    base = (c * NS + s) * NCH
    pltpu.sync_copy(zeros_hbm.at[pl.ds(s * ZR, ZR)], zbuf)
    pltpu.sync_copy(zbuf, acc.at[pl.ds(s * ZR, ZR)])
    plsc.subcore_barrier()
    pltpu.sync_copy(dst_hbm.at[pl.ds(base, NCH)], idx_v)
    pltpu.sync_copy(msg_hbm.at[pl.ds(base, NCH)], rows_v)
    descs = [pltpu.async_copy(rows_v.at[j], acc.at[idx_v.at[j]], sem, add=True)
             for j in range(NCH)]
    for d in descs:
        d.wait()
    plsc.subcore_barrier()
    pltpu.sync_copy(acc.at[pl.ds(s * ZR, ZR)], zbuf)
    pltpu.sync_copy(zbuf, out_hbm.at[c, pl.ds(s * ZR, ZR)])


def _count_body(dst_hbm, zeros_hbm, ones_hbm, out_hbm, idx_v, obuf, zbuf,
                sem, acc):
    c = lax.axis_index("c")
    s = lax.axis_index("s")
    base = (c * NS + s) * NCH
    pltpu.sync_copy(zeros_hbm.at[pl.ds(s * ZR, ZR)], zbuf)
    pltpu.sync_copy(zbuf, acc.at[pl.ds(s * ZR, ZR)])
    pltpu.sync_copy(ones_hbm, obuf)
    plsc.subcore_barrier()
    pltpu.sync_copy(dst_hbm.at[pl.ds(base, NCH)], idx_v)
    descs = [pltpu.async_copy(obuf, acc.at[idx_v.at[j]], sem, add=True)
             for j in range(NCH)]
    for d in descs:
        d.wait()
    plsc.subcore_barrier()
    pltpu.sync_copy(acc.at[pl.ds(s * ZR, ZR)], zbuf)
    pltpu.sync_copy(zbuf, out_hbm.at[c, pl.ds(s * ZR, ZR)])


import functools


@functools.cache
def _sc_kernels():
    mesh = plsc.VectorSubcoreMesh(
        core_axis_name="c", subcore_axis_name="s",
        num_cores=NC, num_subcores=NS)
    params = pltpu.CompilerParams(use_tc_tiling_on_sc=False)
    gather = pl.kernel(
        _gather_body,
        out_type=jax.ShapeDtypeStruct((EPAD // CH, CH, H), _f32),
        mesh=mesh,
        scratch_types=[
            pltpu.VMEM((NCH, CH), jnp.int32),
            pltpu.VMEM((NCH, CH, H), _f32),
            pltpu.SemaphoreType.DMA,
        ],
        compiler_params=params,
    )
    scatter = pl.kernel(
        _scatter_body,
        out_type=jax.ShapeDtypeStruct((NC, NPAD, H), _f32),
        mesh=mesh,
        scratch_types=[
            pltpu.VMEM((NCH, CH), jnp.int32),
            pltpu.VMEM((NCH, CH, H), _f32),
            pltpu.VMEM((ZR, H), _f32),
            pltpu.SemaphoreType.DMA,
            pltpu.VMEM_SHARED((NPAD, H), _f32),
        ],
        compiler_params=params,
    )
    count = pl.kernel(
        _count_body,
        out_type=jax.ShapeDtypeStruct((NC, NPAD, H), _f32),
        mesh=mesh,
        scratch_types=[
            pltpu.VMEM((NCH, CH), jnp.int32),
            pltpu.VMEM((CH, H), _f32),
            pltpu.VMEM((ZR, H), _f32),
            pltpu.SemaphoreType.DMA,
            pltpu.VMEM_SHARED((NPAD, H), _f32),
        ],
        compiler_params=params,
    )
    return gather, scatter, count


TN = 2048
TD = 2000

def _h0_body(x_ref, w_ref, b_ref, o_ref):
    o_ref[...] = jnp.maximum(
        jnp.dot(x_ref[...], w_ref[...], preferred_element_type=_f32)
        + b_ref[...], 0.0)


_h0 = pl.pallas_call(
    _h0_body,
    grid=(N // TD,),
    in_specs=[
        pl.BlockSpec((TD, FN), lambda i: (i, 0)),
        pl.BlockSpec((FN, H), lambda i: (0, 0)),
        pl.BlockSpec((1, H), lambda i: (0, 0)),
    ],
    out_specs=pl.BlockSpec((TD, H), lambda i: (i, 0)),
    out_shape=jax.ShapeDtypeStruct((NPAD, H), _f32),
)


TE = 2000

def _edge_body(ea_ref, g_ref, we1_ref, be1_ref, we2_ref, be2_ref, r_ref,
               s_ref, o_ref):
    u = jnp.maximum(
        jnp.dot(ea_ref[...], we1_ref[...], preferred_element_type=_f32)
        + be1_ref[...], 0.0)
    w = jnp.dot(u, we2_ref[...], preferred_element_type=_f32) + be2_ref[...]
    ge = jnp.dot(g_ref[...], r_ref[...], preferred_element_type=_f32)
    o_ref[...] = jnp.dot(ge * w, s_ref[...], preferred_element_type=_f32)


_edge = pl.pallas_call(
    _edge_body,
    grid=(E // TE,),
    in_specs=[
        pl.BlockSpec((TE, FE), lambda i: (i, 0)),
        pl.BlockSpec((TE, H), lambda i: (i, 0)),
        pl.BlockSpec((FE, H), lambda i: (0, 0)),
        pl.BlockSpec((1, H), lambda i: (0, 0)),
        pl.BlockSpec((H, H * H), lambda i: (0, 0)),
        pl.BlockSpec((1, H * H), lambda i: (0, 0)),
        pl.BlockSpec((H, H * H), lambda i: (0, 0)),
        pl.BlockSpec((H * H, H), lambda i: (0, 0)),
    ],
    out_specs=pl.BlockSpec((TE, H), lambda i: (i, 0)),
    out_shape=jax.ShapeDtypeStruct((EPAD, H), _f32),
)


def _node_body(p0_ref, p1_ref, c0_ref, c1_ref, h_ref, root_ref, b_ref, o_ref):
    cw = jnp.maximum(c0_ref[...][0] + c1_ref[...][0], 1.0)
    agg = (p0_ref[...][0] + p1_ref[...][0]) / cw
    o_ref[...] = jnp.maximum(
        agg + jnp.dot(h_ref[...], root_ref[...], preferred_element_type=_f32)
        + b_ref[...], 0.0)


_node = pl.pallas_call(
    _node_body,
    grid=(NPAD // TN,),
    in_specs=[
        pl.BlockSpec((1, TN, H), lambda i: (0, i, 0)),
        pl.BlockSpec((1, TN, H), lambda i: (1, i, 0)),
        pl.BlockSpec((1, TN, H), lambda i: (0, i, 0)),
        pl.BlockSpec((1, TN, H), lambda i: (1, i, 0)),
        pl.BlockSpec((TN, H), lambda i: (i, 0)),
        pl.BlockSpec((H, H), lambda i: (0, 0)),
        pl.BlockSpec((1, H), lambda i: (0, 0)),
    ],
    out_specs=pl.BlockSpec((TN, H), lambda i: (i, 0)),
    out_shape=jax.ShapeDtypeStruct((NPAD, H), _f32),
)


def _dec_body(h_ref, wo_ref, bo_ref, w1_ref, b1_ref, w2_ref, b2_ref, o_ref):
    z = (jnp.dot(h_ref[...], wo_ref[...], preferred_element_type=_f32)
         + bo_ref[...])
    d = jnp.maximum(
        jnp.dot(z, w1_ref[...], preferred_element_type=_f32) + b1_ref[...],
        0.0)
    o_ref[...] = (jnp.dot(d, w2_ref[...], preferred_element_type=_f32)
                  + b2_ref[...])


_dec = pl.pallas_call(
    _dec_body,
    grid=(N // TD,),
    in_specs=[
        pl.BlockSpec((TD, H), lambda i: (i, 0)),
        pl.BlockSpec((H, Z), lambda i: (0, 0)),
        pl.BlockSpec((1, Z), lambda i: (0, 0)),
        pl.BlockSpec((Z, H), lambda i: (0, 0)),
        pl.BlockSpec((1, H), lambda i: (0, 0)),
        pl.BlockSpec((H, FN), lambda i: (0, 0)),
        pl.BlockSpec((1, FN), lambda i: (0, 0)),
    ],
    out_specs=pl.BlockSpec((TD, FN), lambda i: (i, 0)),
    out_shape=jax.ShapeDtypeStruct((N, FN), _f32),
)


def kernel(x, edge_index, edge_attr, W_in, b_in,
           We1_0, be1_0, We2_0, be2_0, root_0, bconv_0,
           We1_1, be1_1, We2_1, be2_1, root_1, bconv_1,
           We1_2, be1_2, We2_2, be2_2, root_2, bconv_2,
           W_out, b_out, Wd1, bd1, Wd2, bd2):
    src = edge_index[0]
    dst = edge_index[1]
    src_p = jnp.pad(src, (0, EPAD - E)).reshape(EPAD // CH, CH)
    dst_p = jnp.pad(dst, (0, EPAD - E), constant_values=N).reshape(
        EPAD // CH, CH)
    zeros_np = jnp.zeros((NPAD, H), _f32)
    ones_ch = jnp.ones((CH, H), _f32)
    eye = jnp.eye(H, dtype=_f32)
    r_mat = jnp.repeat(eye, H, axis=1)
    s_mat = jnp.tile(eye, (H, 1))

    _gather, _scatter, _count = _sc_kernels()
    h = _h0(x, W_in, b_in.reshape(1, H))
    cnt = _count(dst_p, zeros_np, ones_ch)

    for We1, be1, We2, be2, root, bconv in (
            (We1_0, be1_0, We2_0, be2_0, root_0, bconv_0),
            (We1_1, be1_1, We2_1, be2_1, root_1, bconv_1),
            (We1_2, be1_2, We2_2, be2_2, root_2, bconv_2)):
        g3 = _gather(h, src_p)
        msg = _edge(edge_attr, g3.reshape(EPAD, H), We1, be1.reshape(1, H),
                    We2, be2.reshape(1, H * H), r_mat, s_mat)
        parts = _scatter(msg.reshape(EPAD // CH, CH, H), dst_p, zeros_np)
        h = _node(parts, parts, cnt, cnt, h, root, bconv.reshape(1, H))

    return _dec(h, W_out, b_out.reshape(1, Z), Wd1, bd1.reshape(1, H),
                Wd2, bd2.reshape(1, FN))
